# baseline (speedup 1.0000x reference)
import functools

import jax
import jax.numpy as jnp
from jax import lax
from jax.experimental import pallas as pl
from jax.experimental.pallas import tpu as pltpu
from jax.experimental.pallas import tpu_sc as plsc

NC = 2
NS = 16
NW = NC * NS
CHUNK = 16
BR = 400
LN_EPS = 1e-12


def _make_sc_gather(n_rows, d):
    rows_per_w = n_rows // NW
    n_chunks = rows_per_w // CHUNK

    def body(idx_hbm, word_hbm, out_hbm, idx_v, rows0_v, rows1_v,
             gsem0, gsem1, ssem0, ssem1):
        bufs = (rows0_v, rows1_v)
        gsems = (gsem0, gsem1)
        ssems = (ssem0, ssem1)
        wid = lax.axis_index("s") * NC + lax.axis_index("c")
        base_row = wid * rows_per_w

        pltpu.sync_copy(idx_hbm.at[wid], idx_v)

        def start_gather(c, b):
            pltpu.async_copy(word_hbm.at[idx_v.at[c]], bufs[b], gsems[b])

        def wait_gather(c, b):
            pltpu.make_async_copy(word_hbm.at[idx_v.at[c]], bufs[b],
                                  gsems[b]).wait()

        def start_store(c, b):
            pltpu.async_copy(
                bufs[b], out_hbm.at[pl.ds(base_row + c * CHUNK, CHUNK)],
                ssems[b])

        def wait_store(c, b):
            pltpu.make_async_copy(
                bufs[b], out_hbm.at[pl.ds(base_row + c * CHUNK, CHUNK)],
                ssems[b]).wait()

        start_gather(0, 0)

        def outer(i, _):
            c0 = i * 2
            for b in range(2):
                c = c0 + b

                @pl.when(c + 1 < n_chunks)
                def _():
                    @pl.when(c >= 1)
                    def _():
                        wait_store(c - 1, 1 - b)
                    start_gather(c + 1, 1 - b)

                wait_gather(c, b)
                start_store(c, b)
            return 0

        lax.fori_loop(0, n_chunks // 2, outer, 0)
        wait_store(n_chunks - 2, 0)
        wait_store(n_chunks - 1, 1)

    return pl.kernel(
        body,
        out_type=jax.ShapeDtypeStruct((n_rows, d), jnp.float32),
        mesh=plsc.VectorSubcoreMesh(core_axis_name="c", subcore_axis_name="s"),
        scratch_types=[
            pltpu.VMEM((n_chunks, CHUNK), jnp.int32),
            pltpu.VMEM((CHUNK, d), jnp.float32),
            pltpu.VMEM((CHUNK, d), jnp.float32),
            pltpu.SemaphoreType.DMA,
            pltpu.SemaphoreType.DMA,
            pltpu.SemaphoreType.DMA,
            pltpu.SemaphoreType.DMA,
        ],
    )


def _tc_ln(x_ref, pe_ref, tt_ref, g_ref, b_ref, o_ref):
    h = x_ref[...] + pe_ref[0] + tt_ref[0:1, :]
    mu = jnp.mean(h, axis=1, keepdims=True)
    var = jnp.mean((h - mu) * (h - mu), axis=1, keepdims=True)
    o_ref[...] = ((h - mu) * lax.rsqrt(var + LN_EPS)) * g_ref[0:1, :] \
        + b_ref[0:1, :]


def _tc_ln_acc(x_ref, pe_ref, tt_ref, g_ref, b_ref, prev_ref, o_ref):
    del prev_ref
    _tc_ln(x_ref, pe_ref, tt_ref, g_ref, b_ref, o_ref)


NSLICE = 10


@jax.jit
def _run(x_i32, word_emb, pos_emb, tt_emb, ln_gamma, ln_beta):
    n_seq, seq_len = x_i32.shape
    d = word_emb.shape[1]
    n_rows = n_seq * seq_len

    x_pm = x_i32.T
    pe3 = pos_emb.reshape(seq_len, 1, d)
    g2 = ln_gamma.reshape(1, d)
    b2 = ln_beta.reshape(1, d)

    sl_len = seq_len // NSLICE
    sl_rows = sl_len * n_seq
    sc = _make_sc_gather(sl_rows, d)

    gathered = [
        sc(x_pm[s * sl_len:(s + 1) * sl_len]
           .reshape(NW, sl_rows // NW // CHUNK, CHUNK), word_emb)
        for s in range(NSLICE)
    ]

    out = None
    for s in range(NSLICE):
        off = s * sl_len
        in_specs = [
            pl.BlockSpec((n_seq, d), lambda i: (i, 0)),
            pl.BlockSpec((1, 1, d), lambda i, off=off: (off + i, 0, 0)),
            pl.BlockSpec((2, d), lambda i: (0, 0)),
            pl.BlockSpec((1, d), lambda i: (0, 0)),
            pl.BlockSpec((1, d), lambda i: (0, 0)),
        ]
        out_spec = pl.BlockSpec((n_seq, d), lambda i, off=off: (off + i, 0))
        if s == 0:
            out = pl.pallas_call(
                _tc_ln,
                grid=(sl_len,),
                in_specs=in_specs,
                out_specs=out_spec,
                out_shape=jax.ShapeDtypeStruct((n_rows, d), jnp.float32),
            )(gathered[s], pe3, tt_emb, g2, b2)
        else:
            out = pl.pallas_call(
                _tc_ln_acc,
                grid=(sl_len,),
                in_specs=in_specs + [pl.BlockSpec(memory_space=pl.ANY)],
                out_specs=out_spec,
                out_shape=jax.ShapeDtypeStruct((n_rows, d), jnp.float32),
                input_output_aliases={5: 0},
            )(gathered[s], pe3, tt_emb, g2, b2, out)
    return jnp.transpose(out.reshape(seq_len, n_seq, d), (1, 0, 2))


def kernel(x, word_emb, pos_emb, tt_emb, ln_gamma, ln_beta):
    return _run(x.astype(jnp.int32), word_emb, pos_emb, tt_emb,
                ln_gamma, ln_beta)


# speedup vs baseline: 1.0302x; 1.0302x over previous
import jax
import jax.numpy as jnp
from jax import lax
from jax.experimental import pallas as pl
from jax.experimental.pallas import tpu as pltpu
from jax.experimental.pallas import tpu_sc as plsc

NC = 2
NS = 16
NW = NC * NS
CHUNK = 16
BR = 400
LN_EPS = 1e-12


def _make_sc_gather(n_rows, d):
    rows_per_w = n_rows // NW
    n_chunks = rows_per_w // CHUNK

    def body(idx_hbm, word_hbm, out_hbm, idx_v, rows0_v, rows1_v,
             gsem0, gsem1, ssem0, ssem1):
        bufs = (rows0_v, rows1_v)
        gsems = (gsem0, gsem1)
        ssems = (ssem0, ssem1)
        wid = lax.axis_index("s") * NC + lax.axis_index("c")
        base_row = wid * rows_per_w

        pltpu.sync_copy(idx_hbm.at[wid], idx_v)

        def start_gather(c, b):
            pltpu.async_copy(word_hbm.at[idx_v.at[c]], bufs[b], gsems[b])

        def wait_gather(c, b):
            pltpu.make_async_copy(word_hbm.at[idx_v.at[c]], bufs[b],
                                  gsems[b]).wait()

        def start_store(c, b):
            pltpu.async_copy(
                bufs[b], out_hbm.at[pl.ds(base_row + c * CHUNK, CHUNK)],
                ssems[b])

        def wait_store(c, b):
            pltpu.make_async_copy(
                bufs[b], out_hbm.at[pl.ds(base_row + c * CHUNK, CHUNK)],
                ssems[b]).wait()

        start_gather(0, 0)

        def outer(i, _):
            c0 = i * 2
            for b in range(2):
                c = c0 + b

                @pl.when(c + 1 < n_chunks)
                def _():
                    @pl.when(c >= 1)
                    def _():
                        wait_store(c - 1, 1 - b)
                    start_gather(c + 1, 1 - b)

                wait_gather(c, b)
                start_store(c, b)
            return 0

        lax.fori_loop(0, n_chunks // 2, outer, 0)
        wait_store(n_chunks - 2, 0)
        wait_store(n_chunks - 1, 1)

    return pl.kernel(
        body,
        out_type=jax.ShapeDtypeStruct((n_rows, d), jnp.float32),
        mesh=plsc.VectorSubcoreMesh(core_axis_name="c", subcore_axis_name="s"),
        scratch_types=[
            pltpu.VMEM((n_chunks, CHUNK), jnp.int32),
            pltpu.VMEM((CHUNK, d), jnp.float32),
            pltpu.VMEM((CHUNK, d), jnp.float32),
            pltpu.SemaphoreType.DMA,
            pltpu.SemaphoreType.DMA,
            pltpu.SemaphoreType.DMA,
            pltpu.SemaphoreType.DMA,
        ],
    )


def _tc_ln(x_ref, pe_ref, tt_ref, g_ref, b_ref, o_ref):
    h = x_ref[...] + pe_ref[0] + tt_ref[0:1, :]
    mu = jnp.mean(h, axis=1, keepdims=True)
    var = jnp.mean((h - mu) * (h - mu), axis=1, keepdims=True)
    o_ref[...] = ((h - mu) * lax.rsqrt(var + LN_EPS)) * g_ref[0:1, :] \
        + b_ref[0:1, :]


def _tc_ln_acc(x_ref, pe_ref, tt_ref, g_ref, b_ref, prev_ref, o_ref):
    del prev_ref
    _tc_ln(x_ref, pe_ref, tt_ref, g_ref, b_ref, o_ref)


NSLICE = 5


@jax.jit
def _run(x_i32, word_emb, pos_emb, tt_emb, ln_gamma, ln_beta):
    n_seq, seq_len = x_i32.shape
    d = word_emb.shape[1]
    n_rows = n_seq * seq_len

    x_pm = x_i32.T
    pe3 = pos_emb.reshape(seq_len, 1, d)
    g2 = ln_gamma.reshape(1, d)
    b2 = ln_beta.reshape(1, d)

    sl_len = seq_len // NSLICE
    sl_rows = sl_len * n_seq
    sc = _make_sc_gather(sl_rows, d)

    gathered = [
        sc(x_pm[s * sl_len:(s + 1) * sl_len]
           .reshape(NW, sl_rows // NW // CHUNK, CHUNK), word_emb)
        for s in range(NSLICE)
    ]

    out = None
    for s in range(NSLICE):
        off = s * sl_len
        in_specs = [
            pl.BlockSpec((n_seq, d), lambda i: (i, 0)),
            pl.BlockSpec((1, 1, d), lambda i, off=off: (off + i, 0, 0)),
            pl.BlockSpec((2, d), lambda i: (0, 0)),
            pl.BlockSpec((1, d), lambda i: (0, 0)),
            pl.BlockSpec((1, d), lambda i: (0, 0)),
        ]
        out_spec = pl.BlockSpec((n_seq, d), lambda i, off=off: (off + i, 0))
        if s == 0:
            out = pl.pallas_call(
                _tc_ln,
                grid=(sl_len,),
                in_specs=in_specs,
                out_specs=out_spec,
                out_shape=jax.ShapeDtypeStruct((n_rows, d), jnp.float32),
            )(gathered[s], pe3, tt_emb, g2, b2)
        else:
            out = pl.pallas_call(
                _tc_ln_acc,
                grid=(sl_len,),
                in_specs=in_specs + [pl.BlockSpec(memory_space=pl.ANY)],
                out_specs=out_spec,
                out_shape=jax.ShapeDtypeStruct((n_rows, d), jnp.float32),
                input_output_aliases={5: 0},
            )(gathered[s], pe3, tt_emb, g2, b2, out)
    return jnp.transpose(out.reshape(seq_len, n_seq, d), (1, 0, 2))


def kernel(x, word_emb, pos_emb, tt_emb, ln_gamma, ln_beta):
    return _run(x.astype(jnp.int32), word_emb, pos_emb, tt_emb,
                ln_gamma, ln_beta)
